# baseline (speedup 1.0000x reference)
import functools

import jax
import jax.numpy as jnp
from jax import lax
from jax.experimental import pallas as pl
from jax.experimental.pallas import tpu as pltpu
from jax.experimental.pallas import tpu_sc as plsc

N = 10000
E = 320000
NC = 2
NS = 16
C = 128
NCH = E // C
K = 80
TFULL = NCH // K
KLAST = NCH - TFULL * K
ND = K + 4 - KLAST
KB = K + 4
RPT = 632
NP = RPT * NS
DW = 8
GRID = 2
BR = NP // GRID

_mesh = plsc.VectorSubcoreMesh(core_axis_name="c", subcore_axis_name="s")


def _stage_indices(eic, dummy, idx, w):
  @pl.when(w < TFULL)
  def _():
    pltpu.sync_copy(eic.at[pl.ds(w * K, K)], idx.at[pl.ds(0, K)])
    pltpu.sync_copy(dummy.at[pl.ds(0, 4)], idx.at[pl.ds(K, 4)])

  @pl.when(w == TFULL)
  def _():
    pltpu.sync_copy(eic.at[pl.ds(TFULL * K, KLAST)], idx.at[pl.ds(0, KLAST)])
    pltpu.sync_copy(dummy, idx.at[pl.ds(KLAST, ND)])


def _sc_agg(width):
  @functools.partial(
      pl.kernel,
      out_type=jax.ShapeDtypeStruct((NC, 2, NP, width), jnp.float32),
      mesh=_mesh,
      compiler_params=pltpu.CompilerParams(use_tc_tiling_on_sc=False),
      scratch_types=[
          pltpu.VMEM_SHARED((NP, width), jnp.float32),
          pltpu.VMEM_SHARED((NP, width), jnp.float32),
          pltpu.VMEM_SHARED((NP, width), jnp.float32),
          pltpu.VMEM((KB, 2, C), jnp.int32),
          pltpu.VMEM((C, width), jnp.float32),
          pltpu.VMEM((C, width), jnp.float32),
          pltpu.SemaphoreType.DMA,
          pltpu.SemaphoreType.DMA,
      ],
  )
  def agg(g_hbm, eic, dummy, zeros_hbm, out_hbm,
          acc, acc2, gsp, idx, rows0, rows1, sem0, sem1):
    c = lax.axis_index("c")
    s = lax.axis_index("s")
    w = c * NS + s
    row0 = s * RPT
    pltpu.sync_copy(zeros_hbm.at[pl.ds(row0, RPT)], acc.at[pl.ds(row0, RPT)])
    pltpu.sync_copy(zeros_hbm.at[pl.ds(row0, RPT)], acc2.at[pl.ds(row0, RPT)])
    pltpu.sync_copy(g_hbm.at[pl.ds(row0, RPT)], gsp.at[pl.ds(row0, RPT)])
    _stage_indices(eic, dummy, idx, w)
    plsc.subcore_barrier()

    pltpu.async_copy(gsp.at[idx.at[0, 0]], rows0, sem0)

    def body(j):
      pltpu.async_copy(gsp.at[idx.at[j + 1, 0]], rows1, sem1)
      pltpu.make_async_copy(gsp.at[idx.at[j, 0]], rows0, sem0).wait()
      pltpu.sync_copy(rows0, acc.at[idx.at[j, 1]], add=True)
      pltpu.async_copy(gsp.at[idx.at[j + 2, 0]], rows0, sem0)
      pltpu.make_async_copy(gsp.at[idx.at[j + 1, 0]], rows1, sem1).wait()
      pltpu.sync_copy(rows1, acc2.at[idx.at[j + 1, 1]], add=True)

    pl.loop(0, K, step=2)(body)
    pltpu.make_async_copy(gsp.at[idx.at[0, 0]], rows0, sem0).wait()

    plsc.subcore_barrier()
    pltpu.sync_copy(acc.at[pl.ds(row0, RPT)],
                    out_hbm.at[c, 0, pl.ds(row0, RPT)])
    pltpu.sync_copy(acc2.at[pl.ds(row0, RPT)],
                    out_hbm.at[c, 1, pl.ds(row0, RPT)])

  return agg


def _sc_degree():
  @functools.partial(
      pl.kernel,
      out_type=jax.ShapeDtypeStruct((NC, NP, DW), jnp.float32),
      mesh=_mesh,
      compiler_params=pltpu.CompilerParams(use_tc_tiling_on_sc=False),
      scratch_types=[
          pltpu.VMEM_SHARED((NP, DW), jnp.float32),
          pltpu.VMEM((KB, 2, C), jnp.int32),
          pltpu.VMEM((C, DW), jnp.float32),
      ],
  )
  def deg(eic, dummy, ones_hbm, zeros_hbm, out_hbm, acc, idx, ones_v):
    c = lax.axis_index("c")
    s = lax.axis_index("s")
    w = c * NS + s
    row0 = s * RPT
    pltpu.sync_copy(zeros_hbm.at[pl.ds(row0, RPT)], acc.at[pl.ds(row0, RPT)])
    pltpu.sync_copy(ones_hbm, ones_v)
    _stage_indices(eic, dummy, idx, w)
    plsc.subcore_barrier()

    def body(j):
      pltpu.sync_copy(ones_v, acc.at[idx.at[j, 1]], add=True)

    pl.loop(0, K)(body)

    plsc.subcore_barrier()
    pltpu.sync_copy(acc.at[pl.ds(row0, RPT)],
                    out_hbm.at[c, pl.ds(row0, RPT)])

  return deg


def _tc_matmul1(x_pad, w1):
  def body(x_ref, w_ref, o_ref):
    o_ref[...] = jnp.dot(x_ref[...], w_ref[...],
                         preferred_element_type=jnp.float32)
  return pl.pallas_call(
      body,
      grid=(GRID,),
      in_specs=[pl.BlockSpec((BR, 128), lambda i: (i, 0)),
                pl.BlockSpec((128, 16), lambda i: (0, 0))],
      out_specs=pl.BlockSpec((BR, 16), lambda i: (i, 0)),
      out_shape=jax.ShapeDtypeStruct((NP, 16), jnp.float32),
  )(x_pad, w1)


def _tc_scale(degp, h1):
  def body(d_ref, h_ref, g_ref, dinv_ref):
    deg = (d_ref[0] + d_ref[1])[:, 0:1] + 1.0
    dinv = lax.rsqrt(jnp.maximum(deg, 1.0))
    dinv_ref[...] = dinv
    g_ref[...] = h_ref[...] * dinv
  return pl.pallas_call(
      body,
      grid=(GRID,),
      in_specs=[pl.BlockSpec((2, BR, DW), lambda i: (0, i, 0)),
                pl.BlockSpec((BR, 16), lambda i: (i, 0))],
      out_specs=(pl.BlockSpec((BR, 16), lambda i: (i, 0)),
                 pl.BlockSpec((BR, 1), lambda i: (i, 0))),
      out_shape=(jax.ShapeDtypeStruct((NP, 16), jnp.float32),
                 jax.ShapeDtypeStruct((NP, 1), jnp.float32)),
  )(degp, h1)


def _tc_layer1_finish(p1, g1, dinv, b1_row, w2_pad):
  def body(p_ref, g_ref, dinv_ref, b_ref, w_ref, o_ref):
    acc = p_ref[0] + p_ref[1] + p_ref[2] + p_ref[3] + g_ref[...]
    s = jnp.maximum(acc * dinv_ref[...] + b_ref[...], 0.0)
    h2 = jnp.dot(s, w_ref[...], preferred_element_type=jnp.float32)
    o_ref[...] = h2 * dinv_ref[...]
  return pl.pallas_call(
      body,
      grid=(GRID,),
      in_specs=[pl.BlockSpec((4, BR, 16), lambda i: (0, i, 0)),
                pl.BlockSpec((BR, 16), lambda i: (i, 0)),
                pl.BlockSpec((BR, 1), lambda i: (i, 0)),
                pl.BlockSpec((1, 16), lambda i: (0, 0)),
                pl.BlockSpec((16, 8), lambda i: (0, 0))],
      out_specs=pl.BlockSpec((BR, 8), lambda i: (i, 0)),
      out_shape=jax.ShapeDtypeStruct((NP, 8), jnp.float32),
  )(p1, g1, dinv, b1_row, w2_pad)


def _tc_layer2_finish(p2, g2, dinv, b2_row):
  def body(p_ref, g_ref, dinv_ref, b_ref, o_ref):
    acc = p_ref[0] + p_ref[1] + p_ref[2] + p_ref[3] + g_ref[...]
    o = acc * dinv_ref[...] + b_ref[...]
    col = lax.broadcasted_iota(jnp.int32, o.shape, 1)
    o = jnp.where(col < 7, o, -jnp.inf)
    m = jnp.max(o, axis=1, keepdims=True)
    sh = o - m
    lse = jnp.log(jnp.sum(jnp.exp(sh), axis=1, keepdims=True))
    o_ref[...] = sh - lse
  return pl.pallas_call(
      body,
      grid=(GRID,),
      in_specs=[pl.BlockSpec((4, BR, 8), lambda i: (0, i, 0)),
                pl.BlockSpec((BR, 8), lambda i: (i, 0)),
                pl.BlockSpec((BR, 1), lambda i: (i, 0)),
                pl.BlockSpec((1, 8), lambda i: (0, 0))],
      out_specs=pl.BlockSpec((BR, 8), lambda i: (i, 0)),
      out_shape=jax.ShapeDtypeStruct((NP, 8), jnp.float32),
  )(p2, g2, dinv, b2_row)


def kernel(x, edge_index, W1, b1, W2, b2):
  eic = edge_index.reshape(2, NCH, C).transpose(1, 0, 2)
  dummy = jnp.full((ND, 2, C), N, dtype=jnp.int32)

  zeros16 = jnp.zeros((NP, 16), jnp.float32)
  zeros8 = jnp.zeros((NP, 8), jnp.float32)
  zerosd = jnp.zeros((NP, DW), jnp.float32)
  onesd = jnp.ones((C, DW), jnp.float32)

  x_pad = jnp.pad(x, ((0, NP - N), (0, 0)))
  w2_pad = jnp.pad(W2, ((0, 0), (0, 1)))
  b1_row = b1.reshape(1, 16)
  b2_row = jnp.pad(b2, (0, 1)).reshape(1, 8)

  degp = _sc_degree()(eic, dummy, onesd, zerosd)
  h1 = _tc_matmul1(x_pad, W1)
  g1, dinv = _tc_scale(degp, h1)
  p1 = _sc_agg(16)(g1, eic, dummy, zeros16).reshape(4, NP, 16)
  g2 = _tc_layer1_finish(p1, g1, dinv, b1_row, w2_pad)
  p2 = _sc_agg(8)(g2, eic, dummy, zeros8).reshape(4, NP, 8)
  out = _tc_layer2_finish(p2, g2, dinv, b2_row)
  return out[:N, :7]


# speedup vs baseline: 1.1452x; 1.1452x over previous
import functools

import jax
import jax.numpy as jnp
from jax import lax
from jax.experimental import pallas as pl
from jax.experimental.pallas import tpu as pltpu
from jax.experimental.pallas import tpu_sc as plsc

N = 10000
E = 320000
NC = 2
NS = 16
C = 128
NCH = E // C
K = 80
TFULL = NCH // K
KLAST = NCH - TFULL * K
ND = K + 4 - KLAST
KB = K + 4
RPT = 632
NP = RPT * NS
DW = 8
GRID = 2
BR = NP // GRID

_mesh = plsc.VectorSubcoreMesh(core_axis_name="c", subcore_axis_name="s")


def _stage_indices(eic, dummy, idx, w):
  @pl.when(w < TFULL)
  def _():
    pltpu.sync_copy(eic.at[pl.ds(w * K, K)], idx.at[pl.ds(0, K)])
    pltpu.sync_copy(dummy.at[pl.ds(0, 4)], idx.at[pl.ds(K, 4)])

  @pl.when(w == TFULL)
  def _():
    pltpu.sync_copy(eic.at[pl.ds(TFULL * K, KLAST)], idx.at[pl.ds(0, KLAST)])
    pltpu.sync_copy(dummy, idx.at[pl.ds(KLAST, ND)])


def _sc_agg(width):
  @functools.partial(
      pl.kernel,
      out_type=jax.ShapeDtypeStruct((NC, NP, width), jnp.float32),
      mesh=_mesh,
      compiler_params=pltpu.CompilerParams(use_tc_tiling_on_sc=False),
      scratch_types=[
          pltpu.VMEM_SHARED((NP, width), jnp.float32),
          pltpu.VMEM_SHARED((NP, width), jnp.float32),
          pltpu.VMEM((KB, 2, C), jnp.int32),
          pltpu.VMEM((C, width), jnp.float32),
          pltpu.VMEM((C, width), jnp.float32),
          pltpu.SemaphoreType.DMA,
          pltpu.SemaphoreType.DMA,
      ],
  )
  def agg(g_hbm, eic, dummy, zeros_hbm, out_hbm,
          acc, gsp, idx, rows0, rows1, sem0, sem1):
    c = lax.axis_index("c")
    s = lax.axis_index("s")
    w = c * NS + s
    row0 = s * RPT
    pltpu.sync_copy(zeros_hbm.at[pl.ds(row0, RPT)], acc.at[pl.ds(row0, RPT)])
    pltpu.sync_copy(g_hbm.at[pl.ds(row0, RPT)], gsp.at[pl.ds(row0, RPT)])
    _stage_indices(eic, dummy, idx, w)
    plsc.subcore_barrier()

    pltpu.async_copy(gsp.at[idx.at[0, 0]], rows0, sem0)

    def body(j):
      pltpu.async_copy(gsp.at[idx.at[j + 1, 0]], rows1, sem1)
      pltpu.make_async_copy(gsp.at[idx.at[j, 0]], rows0, sem0).wait()
      pltpu.sync_copy(rows0, acc.at[idx.at[j, 1]], add=True)
      pltpu.async_copy(gsp.at[idx.at[j + 2, 0]], rows0, sem0)
      pltpu.make_async_copy(gsp.at[idx.at[j + 1, 0]], rows1, sem1).wait()
      pltpu.sync_copy(rows1, acc.at[idx.at[j + 1, 1]], add=True)

    pl.loop(0, K, step=2)(body)
    pltpu.make_async_copy(gsp.at[idx.at[0, 0]], rows0, sem0).wait()

    plsc.subcore_barrier()
    pltpu.sync_copy(acc.at[pl.ds(row0, RPT)],
                    out_hbm.at[c, pl.ds(row0, RPT)])

  return agg


def _sc_degree():
  @functools.partial(
      pl.kernel,
      out_type=jax.ShapeDtypeStruct((NC, NP, DW), jnp.float32),
      mesh=_mesh,
      compiler_params=pltpu.CompilerParams(use_tc_tiling_on_sc=False),
      scratch_types=[
          pltpu.VMEM_SHARED((NP, DW), jnp.float32),
          pltpu.VMEM((KB, 2, C), jnp.int32),
          pltpu.VMEM((C, DW), jnp.float32),
      ],
  )
  def deg(eic, dummy, ones_hbm, zeros_hbm, out_hbm, acc, idx, ones_v):
    c = lax.axis_index("c")
    s = lax.axis_index("s")
    w = c * NS + s
    row0 = s * RPT
    pltpu.sync_copy(zeros_hbm.at[pl.ds(row0, RPT)], acc.at[pl.ds(row0, RPT)])
    pltpu.sync_copy(ones_hbm, ones_v)
    _stage_indices(eic, dummy, idx, w)
    plsc.subcore_barrier()

    def body(j):
      pltpu.sync_copy(ones_v, acc.at[idx.at[j, 1]], add=True)

    pl.loop(0, K)(body)

    plsc.subcore_barrier()
    pltpu.sync_copy(acc.at[pl.ds(row0, RPT)],
                    out_hbm.at[c, pl.ds(row0, RPT)])

  return deg


def _tc_matmul1(x_pad, w1):
  def body(x_ref, w_ref, o_ref):
    o_ref[...] = jnp.dot(x_ref[...], w_ref[...],
                         preferred_element_type=jnp.float32)
  return pl.pallas_call(
      body,
      grid=(GRID,),
      in_specs=[pl.BlockSpec((BR, 128), lambda i: (i, 0)),
                pl.BlockSpec((128, 16), lambda i: (0, 0))],
      out_specs=pl.BlockSpec((BR, 16), lambda i: (i, 0)),
      out_shape=jax.ShapeDtypeStruct((NP, 16), jnp.float32),
  )(x_pad, w1)


def _tc_scale(degp, h1):
  def body(d_ref, h_ref, g_ref, dinv_ref):
    deg = (d_ref[0] + d_ref[1])[:, 0:1] + 1.0
    dinv = lax.rsqrt(jnp.maximum(deg, 1.0))
    dinv_ref[...] = dinv
    g_ref[...] = h_ref[...] * dinv
  return pl.pallas_call(
      body,
      grid=(GRID,),
      in_specs=[pl.BlockSpec((2, BR, DW), lambda i: (0, i, 0)),
                pl.BlockSpec((BR, 16), lambda i: (i, 0))],
      out_specs=(pl.BlockSpec((BR, 16), lambda i: (i, 0)),
                 pl.BlockSpec((BR, 1), lambda i: (i, 0))),
      out_shape=(jax.ShapeDtypeStruct((NP, 16), jnp.float32),
                 jax.ShapeDtypeStruct((NP, 1), jnp.float32)),
  )(degp, h1)


def _tc_layer1_finish(p1, g1, dinv, b1_row, w2_pad):
  def body(p_ref, g_ref, dinv_ref, b_ref, w_ref, o_ref):
    acc = p_ref[0] + p_ref[1] + g_ref[...]
    s = jnp.maximum(acc * dinv_ref[...] + b_ref[...], 0.0)
    h2 = jnp.dot(s, w_ref[...], preferred_element_type=jnp.float32)
    o_ref[...] = h2 * dinv_ref[...]
  return pl.pallas_call(
      body,
      grid=(GRID,),
      in_specs=[pl.BlockSpec((2, BR, 16), lambda i: (0, i, 0)),
                pl.BlockSpec((BR, 16), lambda i: (i, 0)),
                pl.BlockSpec((BR, 1), lambda i: (i, 0)),
                pl.BlockSpec((1, 16), lambda i: (0, 0)),
                pl.BlockSpec((16, 8), lambda i: (0, 0))],
      out_specs=pl.BlockSpec((BR, 8), lambda i: (i, 0)),
      out_shape=jax.ShapeDtypeStruct((NP, 8), jnp.float32),
  )(p1, g1, dinv, b1_row, w2_pad)


def _tc_layer2_finish(p2, g2, dinv, b2_row):
  def body(p_ref, g_ref, dinv_ref, b_ref, o_ref):
    acc = p_ref[0] + p_ref[1] + g_ref[...]
    o = acc * dinv_ref[...] + b_ref[...]
    col = lax.broadcasted_iota(jnp.int32, o.shape, 1)
    o = jnp.where(col < 7, o, -jnp.inf)
    m = jnp.max(o, axis=1, keepdims=True)
    sh = o - m
    lse = jnp.log(jnp.sum(jnp.exp(sh), axis=1, keepdims=True))
    o_ref[...] = sh - lse
  return pl.pallas_call(
      body,
      grid=(GRID,),
      in_specs=[pl.BlockSpec((2, BR, 8), lambda i: (0, i, 0)),
                pl.BlockSpec((BR, 8), lambda i: (i, 0)),
                pl.BlockSpec((BR, 1), lambda i: (i, 0)),
                pl.BlockSpec((1, 8), lambda i: (0, 0))],
      out_specs=pl.BlockSpec((BR, 8), lambda i: (i, 0)),
      out_shape=jax.ShapeDtypeStruct((NP, 8), jnp.float32),
  )(p2, g2, dinv, b2_row)


def kernel(x, edge_index, W1, b1, W2, b2):
  eic = edge_index.reshape(2, NCH, C).transpose(1, 0, 2)
  dummy = jnp.full((ND, 2, C), N, dtype=jnp.int32)

  zeros16 = jnp.zeros((NP, 16), jnp.float32)
  zeros8 = jnp.zeros((NP, 8), jnp.float32)
  zerosd = jnp.zeros((NP, DW), jnp.float32)
  onesd = jnp.ones((C, DW), jnp.float32)

  x_pad = jnp.pad(x, ((0, NP - N), (0, 0)))
  w2_pad = jnp.pad(W2, ((0, 0), (0, 1)))
  b1_row = b1.reshape(1, 16)
  b2_row = jnp.pad(b2, (0, 1)).reshape(1, 8)

  degp = _sc_degree()(eic, dummy, onesd, zerosd)
  h1 = _tc_matmul1(x_pad, W1)
  g1, dinv = _tc_scale(degp, h1)
  p1 = _sc_agg(16)(g1, eic, dummy, zeros16)
  g2 = _tc_layer1_finish(p1, g1, dinv, b1_row, w2_pad)
  p2 = _sc_agg(8)(g2, eic, dummy, zeros8)
  out = _tc_layer2_finish(p2, g2, dinv, b2_row)
  return out[:N, :7]
